# baseline (speedup 1.0000x reference)
import jax
import jax.numpy as jnp
from jax.experimental import pallas as pl
from jax.experimental.pallas import tpu as pltpu

CHUNK = 144
NH = 4
RED = 4


def _conv2d(x, w, b=None, stride=1, pad=0):
    out = jax.lax.conv_general_dilated(
        x, w, (stride, stride), [(pad, pad), (pad, pad)],
        dimension_numbers=('NCHW', 'OIHW', 'NCHW'))
    if b is not None:
        out = out + b[None, :, None, None]
    return out


def _lsh(key, x, hash_buckets):
    N, L, C = x.shape
    rot = jax.random.normal(key, (1, C, NH, hash_buckets // 2), dtype=x.dtype)
    rot = jnp.broadcast_to(rot, (N, C, NH, hash_buckets // 2))
    rv = jnp.einsum('btf,bfhi->bhti', x, rot)
    rv = jnp.concatenate([rv, -rv], axis=-1)
    codes = jnp.argmax(rv, axis=-1)
    offsets = (jnp.arange(NH) * hash_buckets).reshape(1, -1, 1)
    return (codes + offsets).reshape(N, -1)


def _add_adj(x):
    back = jnp.concatenate([x[:, :, -1:], x[:, :, :-1]], axis=2)
    fwd = jnp.concatenate([x[:, :, 1:], x[:, :, :1]], axis=2)
    return jnp.concatenate([x, back, fwd], axis=3)


def _dot2_kern(sc_r, y3_r, ret_r):
    ret_r[0] = jax.lax.dot_general(
        sc_r[0].astype(jnp.bfloat16), y3_r[0].astype(jnp.bfloat16),
        (((1,), (0,)), ((), ())), preferred_element_type=jnp.float32)


def _attn_block(xa, ya):
    N, H, nc, _, C = xa.shape
    CRED = ya.shape[-1]
    nrm = jnp.sqrt(jnp.sum(xa * xa, axis=-1, keepdims=True))
    xm = xa / jnp.maximum(nrm, 5e-5)
    xm = _add_adj(xm)
    ya3 = _add_adj(ya)
    raw = jnp.einsum('bhkie,bhkje->bhkij', xa, xm)
    bs = jax.scipy.special.logsumexp(raw, axis=-1, keepdims=True)
    sc = jnp.exp(raw - bs)
    M = N * H * nc
    ret = pl.pallas_call(
        _dot2_kern,
        grid=(M,),
        in_specs=[pl.BlockSpec((1, CHUNK, 3 * CHUNK), lambda i: (i, 0, 0)),
                  pl.BlockSpec((1, 3 * CHUNK, CRED), lambda i: (i, 0, 0))],
        out_specs=pl.BlockSpec((1, CHUNK, CRED), lambda i: (i, 0, 0)),
        out_shape=jax.ShapeDtypeStruct((M, CHUNK, CRED), jnp.float32),
    )(sc.reshape(M, CHUNK, 3 * CHUNK), ya3.reshape(M, 3 * CHUNK, CRED))
    return ret.reshape(N, H, nc, CHUNK, CRED), bs


def _nlsa(x, cm_w, cm_b, ca_w, ca_b, key):
    N, Cin, H, W = x.shape
    L = H * W
    xe = _conv2d(x, cm_w, cm_b, 1, 1).reshape(N, -1, L).transpose(0, 2, 1)
    ye = _conv2d(x, ca_w, ca_b, 1, 0).reshape(N, -1, L).transpose(0, 2, 1)
    C = xe.shape[-1]
    hb = min(L // CHUNK + (L // CHUNK) % 2, 128)
    codes = jax.lax.stop_gradient(_lsh(key, xe, hb))
    idx = jnp.argsort(codes, axis=-1)
    undo = jnp.argsort(idx, axis=-1)
    mi = idx % L
    xs = jnp.take_along_axis(xe, mi[:, :, None], axis=1)
    ys = jnp.take_along_axis(ye, mi[:, :, None], axis=1)
    pad = (CHUNK - L % CHUNK) if (L % CHUNK != 0) else 0
    xa = xs.reshape(N, NH, L, C)
    ya = ys.reshape(N, NH, L, C * RED)
    if pad:
        xa = jnp.concatenate([xa, xa[:, :, -pad:, :]], axis=2)
        ya = jnp.concatenate([ya, ya[:, :, -pad:, :]], axis=2)
    nc = xa.shape[2] // CHUNK
    xa = xa.reshape(N, NH, nc, CHUNK, C)
    ya = ya.reshape(N, NH, nc, CHUNK, C * RED)
    ret, bs = _attn_block(xa, ya)
    ret = ret.reshape(N, NH, -1, C * RED)
    bs = bs.reshape(N, NH, -1)
    if pad:
        ret = ret[:, :, :-pad]
        bs = bs[:, :, :-pad]
    ret = ret.reshape(N, -1, C * RED)
    bs = bs.reshape(N, -1)
    ret = jnp.take_along_axis(ret, undo[:, :, None], axis=1)
    bs = jnp.take_along_axis(bs, undo, axis=1)
    ret = ret.reshape(N, NH, L, C * RED)
    bs = bs.reshape(N, NH, L, 1)
    probs = jax.nn.softmax(bs, axis=1)
    ret = jnp.sum(ret * probs, axis=1)
    return ret.transpose(0, 2, 1).reshape(N, -1, H, W) + x


def kernel(x, params):
    rk = jax.random.key(42)
    fea = jax.nn.leaky_relu(_conv2d(x, params['in_proj_w'], None, 1, 1),
                            negative_slope=0.1)
    for i, st in enumerate(params['stages']):
        fea = _nlsa(fea, st['cm1_w'], st['cm1_b'], st['ca1_w'], st['ca1_b'],
                    jax.random.fold_in(rk, 2 * i))
        fea = _nlsa(fea, st['cm2_w'], st['cm2_b'], st['ca2_w'], st['ca2_b'],
                    jax.random.fold_in(rk, 2 * i + 1))
        fea = _conv2d(fea, st['down_w'], None, 2, 1)
    fea = _conv2d(fea, params['out_proj_w'], None, 1, 0)
    return fea.reshape(fea.shape[0], -1)


# speedup vs baseline: 2.5446x; 2.5446x over previous
import functools

import jax
import jax.numpy as jnp
from jax import lax
from jax.experimental import pallas as pl
from jax.experimental.pallas import tpu as pltpu
from jax.experimental.pallas import tpu_sc as plsc

CHUNK = 144
NH = 4
RED = 4


def _sc_gather(table, idx):
    B = idx.shape[0]
    D = table.shape[1]
    info = plsc.get_sparse_core_info()
    NC, NS = info.num_cores, info.num_subcores
    NW = NC * NS
    b_per_w = B // NW
    cap = max(8, (48 * 1024) // (4 * D))
    ch = 1
    while ch * 2 <= min(b_per_w, cap):
        ch *= 2
    n_iter = b_per_w // ch
    mesh = plsc.VectorSubcoreMesh(core_axis_name="c", subcore_axis_name="s")

    @functools.partial(
        pl.kernel, mesh=mesh,
        out_type=jax.ShapeDtypeStruct((B, D), jnp.float32),
        scratch_types=[pltpu.VMEM((ch,), jnp.int32),
                       pltpu.VMEM((ch, D), jnp.float32),
                       pltpu.SemaphoreType.DMA],
    )
    def k(table_hbm, idx_hbm, out_hbm, idx_v, rows_v, sem):
        wid = lax.axis_index("s") * NC + lax.axis_index("c")
        base = wid * b_per_w
        for i in range(n_iter):
            off = base + i * ch
            pltpu.sync_copy(idx_hbm.at[pl.ds(off, ch)], idx_v)
            pltpu.async_copy(table_hbm.at[idx_v], rows_v, sem).wait()
            pltpu.sync_copy(rows_v, out_hbm.at[pl.ds(off, ch)])

    return k(table, idx)


def _pad_cols(a, mult=128):
    d = a.shape[-1]
    p = (-d) % mult
    if p:
        a = jnp.concatenate([a, jnp.zeros(a.shape[:-1] + (p,), a.dtype)], -1)
    return a


def _conv2d(x, w, b=None, stride=1, pad=0):
    out = jax.lax.conv_general_dilated(
        x, w, (stride, stride), [(pad, pad), (pad, pad)],
        dimension_numbers=('NCHW', 'OIHW', 'NCHW'))
    if b is not None:
        out = out + b[None, :, None, None]
    return out


def _lsh(key, x, hash_buckets):
    N, L, C = x.shape
    rot = jax.random.normal(key, (1, C, NH, hash_buckets // 2), dtype=x.dtype)
    rot = jnp.broadcast_to(rot, (N, C, NH, hash_buckets // 2))
    rv = jnp.einsum('btf,bfhi->bhti', x, rot)
    rv = jnp.concatenate([rv, -rv], axis=-1)
    codes = jnp.argmax(rv, axis=-1)
    offsets = (jnp.arange(NH) * hash_buckets).reshape(1, -1, 1)
    return (codes + offsets).reshape(N, -1)


def _add_adj(x):
    back = jnp.concatenate([x[:, :, -1:], x[:, :, :-1]], axis=2)
    fwd = jnp.concatenate([x[:, :, 1:], x[:, :, :1]], axis=2)
    return jnp.concatenate([x, back, fwd], axis=3)


def _dot2_kern(sc_r, y3_r, ret_r):
    ret_r[0] = jax.lax.dot_general(
        sc_r[0].astype(jnp.bfloat16), y3_r[0].astype(jnp.bfloat16),
        (((1,), (0,)), ((), ())), preferred_element_type=jnp.float32)


def _attn_block(xa, ya):
    N, H, nc, _, C = xa.shape
    CRED = ya.shape[-1]
    nrm = jnp.sqrt(jnp.sum(xa * xa, axis=-1, keepdims=True))
    xm = xa / jnp.maximum(nrm, 5e-5)
    xm = _add_adj(xm)
    ya3 = _add_adj(ya)
    raw = jnp.einsum('bhkie,bhkje->bhkij', xa, xm)
    bs = jax.scipy.special.logsumexp(raw, axis=-1, keepdims=True)
    sc = jnp.exp(raw - bs)
    M = N * H * nc
    ret = pl.pallas_call(
        _dot2_kern,
        grid=(M,),
        in_specs=[pl.BlockSpec((1, CHUNK, 3 * CHUNK), lambda i: (i, 0, 0)),
                  pl.BlockSpec((1, 3 * CHUNK, CRED), lambda i: (i, 0, 0))],
        out_specs=pl.BlockSpec((1, CHUNK, CRED), lambda i: (i, 0, 0)),
        out_shape=jax.ShapeDtypeStruct((M, CHUNK, CRED), jnp.float32),
    )(sc.reshape(M, CHUNK, 3 * CHUNK), ya3.reshape(M, 3 * CHUNK, CRED))
    return ret.reshape(N, H, nc, CHUNK, CRED), bs


def _nlsa(x, cm_w, cm_b, ca_w, ca_b, key):
    N, Cin, H, W = x.shape
    L = H * W
    xe = _conv2d(x, cm_w, cm_b, 1, 1).reshape(N, -1, L).transpose(0, 2, 1)
    ye = _conv2d(x, ca_w, ca_b, 1, 0).reshape(N, -1, L).transpose(0, 2, 1)
    C = xe.shape[-1]
    hb = min(L // CHUNK + (L // CHUNK) % 2, 128)
    codes = jax.lax.stop_gradient(_lsh(key, xe, hb))
    idx = jnp.argsort(codes, axis=-1)
    undo = jnp.argsort(idx, axis=-1)
    mi = idx % L
    xy = _pad_cols(jnp.concatenate([xe, ye], axis=-1))
    Dp = xy.shape[-1]
    gidx = (mi + (jnp.arange(N, dtype=mi.dtype) * L)[:, None]).reshape(-1)
    g = _sc_gather(xy.reshape(N * L, Dp), gidx.astype(jnp.int32))
    g = g.reshape(N, NH * L, Dp)
    xs = g[..., :C]
    ys = g[..., C:C + C * RED]
    pad = (CHUNK - L % CHUNK) if (L % CHUNK != 0) else 0
    xa = xs.reshape(N, NH, L, C)
    ya = ys.reshape(N, NH, L, C * RED)
    if pad:
        xa = jnp.concatenate([xa, xa[:, :, -pad:, :]], axis=2)
        ya = jnp.concatenate([ya, ya[:, :, -pad:, :]], axis=2)
    nc = xa.shape[2] // CHUNK
    xa = xa.reshape(N, NH, nc, CHUNK, C)
    ya = ya.reshape(N, NH, nc, CHUNK, C * RED)
    ret, bs = _attn_block(xa, ya)
    ret = ret.reshape(N, NH, -1, C * RED)
    bs = bs.reshape(N, NH, -1)
    if pad:
        ret = ret[:, :, :-pad]
        bs = bs[:, :, :-pad]
    ret = ret.reshape(N, -1, C * RED)
    bs = bs.reshape(N, -1)
    S = ret.shape[1]
    rb = _pad_cols(jnp.concatenate([ret, bs[..., None]], axis=-1))
    Du = rb.shape[-1]
    uidx = (undo + (jnp.arange(N, dtype=undo.dtype) * S)[:, None]).reshape(-1)
    gu = _sc_gather(rb.reshape(N * S, Du), uidx.astype(jnp.int32))
    gu = gu.reshape(N, S, Du)
    ret = gu[..., :C * RED].reshape(N, NH, L, C * RED)
    bs = gu[..., C * RED].reshape(N, NH, L, 1)
    probs = jax.nn.softmax(bs, axis=1)
    ret = jnp.sum(ret * probs, axis=1)
    return ret.transpose(0, 2, 1).reshape(N, -1, H, W) + x


def kernel(x, params):
    rk = jax.random.key(42)
    fea = jax.nn.leaky_relu(_conv2d(x, params['in_proj_w'], None, 1, 1),
                            negative_slope=0.1)
    for i, st in enumerate(params['stages']):
        fea = _nlsa(fea, st['cm1_w'], st['cm1_b'], st['ca1_w'], st['ca1_b'],
                    jax.random.fold_in(rk, 2 * i))
        fea = _nlsa(fea, st['cm2_w'], st['cm2_b'], st['ca2_w'], st['ca2_b'],
                    jax.random.fold_in(rk, 2 * i + 1))
        fea = _conv2d(fea, st['down_w'], None, 2, 1)
    fea = _conv2d(fea, params['out_proj_w'], None, 1, 0)
    return fea.reshape(fea.shape[0], -1)


# speedup vs baseline: 2.7136x; 1.0664x over previous
import functools

import jax
import jax.numpy as jnp
from jax import lax
from jax.experimental import pallas as pl
from jax.experimental.pallas import tpu as pltpu
from jax.experimental.pallas import tpu_sc as plsc

CHUNK = 144
NH = 4
RED = 4


def _sc_gather(table, idx):
    B = idx.shape[0]
    D = table.shape[1]
    info = plsc.get_sparse_core_info()
    NC, NS = info.num_cores, info.num_subcores
    NW = NC * NS
    b_per_w = B // NW
    cap = max(8, (48 * 1024) // (4 * D))
    ch = 1
    while ch * 2 <= min(b_per_w, cap):
        ch *= 2
    n_iter = b_per_w // ch
    mesh = plsc.VectorSubcoreMesh(core_axis_name="c", subcore_axis_name="s")

    @functools.partial(
        pl.kernel, mesh=mesh,
        out_type=jax.ShapeDtypeStruct((B, D), jnp.float32),
        scratch_types=[pltpu.VMEM((ch,), jnp.int32),
                       pltpu.VMEM((ch, D), jnp.float32),
                       pltpu.SemaphoreType.DMA],
    )
    def k(table_hbm, idx_hbm, out_hbm, idx_v, rows_v, sem):
        wid = lax.axis_index("s") * NC + lax.axis_index("c")
        base = wid * b_per_w
        for i in range(n_iter):
            off = base + i * ch
            pltpu.sync_copy(idx_hbm.at[pl.ds(off, ch)], idx_v)
            pltpu.async_copy(table_hbm.at[idx_v], rows_v, sem).wait()
            pltpu.sync_copy(rows_v, out_hbm.at[pl.ds(off, ch)])

    return k(table, idx)


def _sc_scatter(rows, idx, out_rows):
    B = idx.shape[0]
    D = rows.shape[1]
    info = plsc.get_sparse_core_info()
    NC, NS = info.num_cores, info.num_subcores
    NW = NC * NS
    b_per_w = B // NW
    cap = max(8, (48 * 1024) // (4 * D))
    ch = 1
    while ch * 2 <= min(b_per_w, cap):
        ch *= 2
    n_iter = b_per_w // ch
    mesh = plsc.VectorSubcoreMesh(core_axis_name="c", subcore_axis_name="s")

    @functools.partial(
        pl.kernel, mesh=mesh,
        out_type=jax.ShapeDtypeStruct((out_rows, D), jnp.float32),
        scratch_types=[pltpu.VMEM((ch,), jnp.int32),
                       pltpu.VMEM((ch, D), jnp.float32),
                       pltpu.SemaphoreType.DMA],
    )
    def k(rows_hbm, idx_hbm, out_hbm, idx_v, rows_v, sem):
        wid = lax.axis_index("s") * NC + lax.axis_index("c")
        base = wid * b_per_w
        for i in range(n_iter):
            off = base + i * ch
            pltpu.sync_copy(idx_hbm.at[pl.ds(off, ch)], idx_v)
            pltpu.sync_copy(rows_hbm.at[pl.ds(off, ch)], rows_v)
            pltpu.async_copy(rows_v, out_hbm.at[idx_v], sem).wait()

    return k(rows, idx)


def _pad_cols(a, mult=128):
    d = a.shape[-1]
    p = (-d) % mult
    if p:
        a = jnp.concatenate([a, jnp.zeros(a.shape[:-1] + (p,), a.dtype)], -1)
    return a


def _conv2d(x, w, b=None, stride=1, pad=0):
    out = jax.lax.conv_general_dilated(
        x, w, (stride, stride), [(pad, pad), (pad, pad)],
        dimension_numbers=('NCHW', 'OIHW', 'NCHW'))
    if b is not None:
        out = out + b[None, :, None, None]
    return out


def _lsh(key, x, hash_buckets):
    N, L, C = x.shape
    rot = jax.random.normal(key, (1, C, NH, hash_buckets // 2), dtype=x.dtype)
    rot = jnp.broadcast_to(rot, (N, C, NH, hash_buckets // 2))
    rv = jnp.einsum('btf,bfhi->bhti', x, rot)
    rv = jnp.concatenate([rv, -rv], axis=-1)
    codes = jnp.argmax(rv, axis=-1)
    offsets = (jnp.arange(NH) * hash_buckets).reshape(1, -1, 1)
    return (codes + offsets).reshape(N, -1)


def _add_adj(x):
    back = jnp.concatenate([x[:, :, -1:], x[:, :, :-1]], axis=2)
    fwd = jnp.concatenate([x[:, :, 1:], x[:, :, :1]], axis=2)
    return jnp.concatenate([x, back, fwd], axis=3)


def _dot2_kern(sc_r, y3_r, ret_r):
    ret_r[0] = jax.lax.dot_general(
        sc_r[0].astype(jnp.bfloat16), y3_r[0].astype(jnp.bfloat16),
        (((1,), (0,)), ((), ())), preferred_element_type=jnp.float32)


def _attn_block(xa, ya):
    N, H, nc, _, C = xa.shape
    CRED = ya.shape[-1]
    nrm = jnp.sqrt(jnp.sum(xa * xa, axis=-1, keepdims=True))
    xm = xa / jnp.maximum(nrm, 5e-5)
    xm = _add_adj(xm)
    ya3 = _add_adj(ya)
    raw = jnp.einsum('bhkie,bhkje->bhkij', xa, xm)
    bs = jax.scipy.special.logsumexp(raw, axis=-1, keepdims=True)
    sc = jnp.exp(raw - bs)
    M = N * H * nc
    ret = pl.pallas_call(
        _dot2_kern,
        grid=(M,),
        in_specs=[pl.BlockSpec((1, CHUNK, 3 * CHUNK), lambda i: (i, 0, 0)),
                  pl.BlockSpec((1, 3 * CHUNK, CRED), lambda i: (i, 0, 0))],
        out_specs=pl.BlockSpec((1, CHUNK, CRED), lambda i: (i, 0, 0)),
        out_shape=jax.ShapeDtypeStruct((M, CHUNK, CRED), jnp.float32),
    )(sc.reshape(M, CHUNK, 3 * CHUNK), ya3.reshape(M, 3 * CHUNK, CRED))
    return ret.reshape(N, H, nc, CHUNK, CRED), bs


def _nlsa(x, cm_w, cm_b, ca_w, ca_b, key):
    N, Cin, H, W = x.shape
    L = H * W
    xe = _conv2d(x, cm_w, cm_b, 1, 1).reshape(N, -1, L).transpose(0, 2, 1)
    ye = _conv2d(x, ca_w, ca_b, 1, 0).reshape(N, -1, L).transpose(0, 2, 1)
    C = xe.shape[-1]
    hb = min(L // CHUNK + (L // CHUNK) % 2, 128)
    codes = jax.lax.stop_gradient(_lsh(key, xe, hb))
    idx = jnp.argsort(codes, axis=-1)
    mi = idx % L
    xy = _pad_cols(jnp.concatenate([xe, ye], axis=-1))
    Dp = xy.shape[-1]
    gidx = (mi + (jnp.arange(N, dtype=mi.dtype) * L)[:, None]).reshape(-1)
    g = _sc_gather(xy.reshape(N * L, Dp), gidx.astype(jnp.int32))
    g = g.reshape(N, NH * L, Dp)
    xs = g[..., :C]
    ys = g[..., C:C + C * RED]
    pad = (CHUNK - L % CHUNK) if (L % CHUNK != 0) else 0
    xa = xs.reshape(N, NH, L, C)
    ya = ys.reshape(N, NH, L, C * RED)
    if pad:
        xa = jnp.concatenate([xa, xa[:, :, -pad:, :]], axis=2)
        ya = jnp.concatenate([ya, ya[:, :, -pad:, :]], axis=2)
    nc = xa.shape[2] // CHUNK
    xa = xa.reshape(N, NH, nc, CHUNK, C)
    ya = ya.reshape(N, NH, nc, CHUNK, C * RED)
    ret, bs = _attn_block(xa, ya)
    ret = ret.reshape(N, NH, -1, C * RED)
    bs = bs.reshape(N, NH, -1)
    if pad:
        ret = ret[:, :, :-pad]
        bs = bs[:, :, :-pad]
    ret = ret.reshape(N, -1, C * RED)
    bs = bs.reshape(N, -1)
    S = ret.shape[1]
    rb = _pad_cols(jnp.concatenate([ret, bs[..., None]], axis=-1))
    Du = rb.shape[-1]
    sidx = (idx + (jnp.arange(N, dtype=idx.dtype) * S)[:, None]).reshape(-1)
    gu = _sc_scatter(rb.reshape(N * S, Du), sidx.astype(jnp.int32), N * S)
    gu = gu.reshape(N, S, Du)
    ret = gu[..., :C * RED].reshape(N, NH, L, C * RED)
    bs = gu[..., C * RED].reshape(N, NH, L, 1)
    probs = jax.nn.softmax(bs, axis=1)
    ret = jnp.sum(ret * probs, axis=1)
    return ret.transpose(0, 2, 1).reshape(N, -1, H, W) + x


def kernel(x, params):
    rk = jax.random.key(42)
    fea = jax.nn.leaky_relu(_conv2d(x, params['in_proj_w'], None, 1, 1),
                            negative_slope=0.1)
    for i, st in enumerate(params['stages']):
        fea = _nlsa(fea, st['cm1_w'], st['cm1_b'], st['ca1_w'], st['ca1_b'],
                    jax.random.fold_in(rk, 2 * i))
        fea = _nlsa(fea, st['cm2_w'], st['cm2_b'], st['ca2_w'], st['ca2_b'],
                    jax.random.fold_in(rk, 2 * i + 1))
        fea = _conv2d(fea, st['down_w'], None, 2, 1)
    fea = _conv2d(fea, params['out_proj_w'], None, 1, 0)
    return fea.reshape(fea.shape[0], -1)


# speedup vs baseline: 2.8626x; 1.0549x over previous
import functools

import jax
import jax.numpy as jnp
from jax import lax
from jax.experimental import pallas as pl
from jax.experimental.pallas import tpu as pltpu
from jax.experimental.pallas import tpu_sc as plsc

CHUNK = 144
NH = 4
RED = 4


def _sc_gather(table, idx):
    B = idx.shape[0]
    D = table.shape[1]
    info = plsc.get_sparse_core_info()
    NC, NS = info.num_cores, info.num_subcores
    NW = NC * NS
    b_per_w = B // NW
    cap = max(8, (48 * 1024) // (4 * D))
    ch = 1
    while ch * 2 <= min(b_per_w, cap):
        ch *= 2
    n_iter = b_per_w // ch
    mesh = plsc.VectorSubcoreMesh(core_axis_name="c", subcore_axis_name="s")

    @functools.partial(
        pl.kernel, mesh=mesh,
        out_type=jax.ShapeDtypeStruct((B, D), jnp.float32),
        scratch_types=[pltpu.VMEM((ch,), jnp.int32),
                       pltpu.VMEM((ch, D), jnp.float32),
                       pltpu.SemaphoreType.DMA],
    )
    def k(table_hbm, idx_hbm, out_hbm, idx_v, rows_v, sem):
        wid = lax.axis_index("s") * NC + lax.axis_index("c")
        base = wid * b_per_w
        for i in range(n_iter):
            off = base + i * ch
            pltpu.sync_copy(idx_hbm.at[pl.ds(off, ch)], idx_v)
            pltpu.async_copy(table_hbm.at[idx_v], rows_v, sem).wait()
            pltpu.sync_copy(rows_v, out_hbm.at[pl.ds(off, ch)])

    return k(table, idx)


def _sc_scatter(rows, idx, out_rows):
    B = idx.shape[0]
    D = rows.shape[1]
    info = plsc.get_sparse_core_info()
    NC, NS = info.num_cores, info.num_subcores
    NW = NC * NS
    b_per_w = B // NW
    cap = max(8, (48 * 1024) // (4 * D))
    ch = 1
    while ch * 2 <= min(b_per_w, cap):
        ch *= 2
    n_iter = b_per_w // ch
    mesh = plsc.VectorSubcoreMesh(core_axis_name="c", subcore_axis_name="s")

    @functools.partial(
        pl.kernel, mesh=mesh,
        out_type=jax.ShapeDtypeStruct((out_rows, D), jnp.float32),
        scratch_types=[pltpu.VMEM((ch,), jnp.int32),
                       pltpu.VMEM((ch, D), jnp.float32),
                       pltpu.SemaphoreType.DMA],
    )
    def k(rows_hbm, idx_hbm, out_hbm, idx_v, rows_v, sem):
        wid = lax.axis_index("s") * NC + lax.axis_index("c")
        base = wid * b_per_w
        for i in range(n_iter):
            off = base + i * ch
            pltpu.sync_copy(idx_hbm.at[pl.ds(off, ch)], idx_v)
            pltpu.sync_copy(rows_hbm.at[pl.ds(off, ch)], rows_v)
            pltpu.async_copy(rows_v, out_hbm.at[idx_v], sem).wait()

    return k(rows, idx)


def _pad_cols(a, mult=128):
    d = a.shape[-1]
    p = (-d) % mult
    if p:
        a = jnp.concatenate([a, jnp.zeros(a.shape[:-1] + (p,), a.dtype)], -1)
    return a


def _conv2d(x, w, b=None, stride=1, pad=0):
    out = jax.lax.conv_general_dilated(
        x, w, (stride, stride), [(pad, pad), (pad, pad)],
        dimension_numbers=('NCHW', 'OIHW', 'NCHW'))
    if b is not None:
        out = out + b[None, :, None, None]
    return out


def _lsh(key, x, hash_buckets):
    N, L, C = x.shape
    rot = jax.random.normal(key, (1, C, NH, hash_buckets // 2), dtype=x.dtype)
    rot = jnp.broadcast_to(rot, (N, C, NH, hash_buckets // 2))
    rv = jnp.einsum('btf,bfhi->bhti', x, rot)
    rv = jnp.concatenate([rv, -rv], axis=-1)
    codes = jnp.argmax(rv, axis=-1)
    offsets = (jnp.arange(NH) * hash_buckets).reshape(1, -1, 1)
    return (codes + offsets).reshape(N, -1)


def _add_adj(x):
    back = jnp.concatenate([x[:, :, -1:], x[:, :, :-1]], axis=2)
    fwd = jnp.concatenate([x[:, :, 1:], x[:, :, :1]], axis=2)
    return jnp.concatenate([x, back, fwd], axis=3)


def _attn_kern(q_r, xc_r, xb_r, xf_r, yc_r, yb_r, yf_r, ret_r, bs_r):
    q = q_r[0]
    xm = jnp.concatenate([xc_r[0], xb_r[0], xf_r[0]], axis=0)
    raw = jax.lax.dot_general(q.astype(jnp.bfloat16), xm.astype(jnp.bfloat16),
                              (((1,), (1,)), ((), ())),
                              preferred_element_type=jnp.float32)
    bs = jax.scipy.special.logsumexp(raw, axis=-1, keepdims=True)
    sc = jnp.exp(raw - bs)
    ya3 = jnp.concatenate([yc_r[0], yb_r[0], yf_r[0]], axis=0)
    ret_r[0] = jax.lax.dot_general(sc.astype(jnp.bfloat16),
                                   ya3.astype(jnp.bfloat16),
                                   (((1,), (0,)), ((), ())),
                                   preferred_element_type=jnp.float32)
    bs_r[...] = bs.reshape(1, 1, CHUNK)


def _attn_block(xa, ya):
    N, H, nc, _, C = xa.shape
    CRED = ya.shape[-1]
    G = N * H
    xa = xa.reshape(G, nc, CHUNK, C)
    ya = ya.reshape(G, nc, CHUNK, CRED)
    nrm = jnp.sqrt(jnp.sum(xa * xa, axis=-1, keepdims=True))
    xm = xa / jnp.maximum(nrm, 5e-5)
    xb = jnp.roll(xm, 1, axis=1)
    xf = jnp.roll(xm, -1, axis=1)
    yb = jnp.roll(ya, 1, axis=1)
    yf = jnp.roll(ya, -1, axis=1)
    M = G * nc
    rs = lambda a: a.reshape(M, CHUNK, a.shape[-1])
    spec_x = pl.BlockSpec((1, CHUNK, C), lambda i: (i, 0, 0))
    spec_y = pl.BlockSpec((1, CHUNK, CRED), lambda i: (i, 0, 0))
    ret, bs = pl.pallas_call(
        _attn_kern,
        grid=(M,),
        in_specs=[spec_x, spec_x, spec_x, spec_x, spec_y, spec_y, spec_y],
        out_specs=[pl.BlockSpec((1, CHUNK, CRED), lambda i: (i, 0, 0)),
                   pl.BlockSpec((1, 1, CHUNK), lambda i: (i, 0, 0))],
        out_shape=[jax.ShapeDtypeStruct((M, CHUNK, CRED), jnp.float32),
                   jax.ShapeDtypeStruct((M, 1, CHUNK), jnp.float32)],
    )(rs(xa), rs(xm), rs(xb), rs(xf), rs(ya), rs(yb), rs(yf))
    return (ret.reshape(N, H, nc, CHUNK, CRED),
            bs.reshape(N, H, nc, CHUNK, 1))


def _nlsa(x, cm_w, cm_b, ca_w, ca_b, key):
    N, Cin, H, W = x.shape
    L = H * W
    xe = _conv2d(x, cm_w, cm_b, 1, 1).reshape(N, -1, L).transpose(0, 2, 1)
    ye = _conv2d(x, ca_w, ca_b, 1, 0).reshape(N, -1, L).transpose(0, 2, 1)
    C = xe.shape[-1]
    hb = min(L // CHUNK + (L // CHUNK) % 2, 128)
    codes = jax.lax.stop_gradient(_lsh(key, xe, hb))
    idx = jnp.argsort(codes, axis=-1)
    mi = idx % L
    xy = _pad_cols(jnp.concatenate([xe, ye], axis=-1))
    Dp = xy.shape[-1]
    gidx = (mi + (jnp.arange(N, dtype=mi.dtype) * L)[:, None]).reshape(-1)
    g = _sc_gather(xy.reshape(N * L, Dp), gidx.astype(jnp.int32))
    g = g.reshape(N, NH * L, Dp)
    xs = g[..., :C]
    ys = g[..., C:C + C * RED]
    pad = (CHUNK - L % CHUNK) if (L % CHUNK != 0) else 0
    xa = xs.reshape(N, NH, L, C)
    ya = ys.reshape(N, NH, L, C * RED)
    if pad:
        xa = jnp.concatenate([xa, xa[:, :, -pad:, :]], axis=2)
        ya = jnp.concatenate([ya, ya[:, :, -pad:, :]], axis=2)
    nc = xa.shape[2] // CHUNK
    xa = xa.reshape(N, NH, nc, CHUNK, C)
    ya = ya.reshape(N, NH, nc, CHUNK, C * RED)
    ret, bs = _attn_block(xa, ya)
    ret = ret.reshape(N, NH, -1, C * RED)
    bs = bs.reshape(N, NH, -1)
    if pad:
        ret = ret[:, :, :-pad]
        bs = bs[:, :, :-pad]
    ret = ret.reshape(N, -1, C * RED)
    bs = bs.reshape(N, -1)
    S = ret.shape[1]
    rb = _pad_cols(jnp.concatenate([ret, bs[..., None]], axis=-1))
    Du = rb.shape[-1]
    sidx = (idx + (jnp.arange(N, dtype=idx.dtype) * S)[:, None]).reshape(-1)
    gu = _sc_scatter(rb.reshape(N * S, Du), sidx.astype(jnp.int32), N * S)
    gu = gu.reshape(N, S, Du)
    ret = gu[..., :C * RED].reshape(N, NH, L, C * RED)
    bs = gu[..., C * RED].reshape(N, NH, L, 1)
    probs = jax.nn.softmax(bs, axis=1)
    ret = jnp.sum(ret * probs, axis=1)
    return ret.transpose(0, 2, 1).reshape(N, -1, H, W) + x


def kernel(x, params):
    rk = jax.random.key(42)
    fea = jax.nn.leaky_relu(_conv2d(x, params['in_proj_w'], None, 1, 1),
                            negative_slope=0.1)
    for i, st in enumerate(params['stages']):
        fea = _nlsa(fea, st['cm1_w'], st['cm1_b'], st['ca1_w'], st['ca1_b'],
                    jax.random.fold_in(rk, 2 * i))
        fea = _nlsa(fea, st['cm2_w'], st['cm2_b'], st['ca2_w'], st['ca2_b'],
                    jax.random.fold_in(rk, 2 * i + 1))
        fea = _conv2d(fea, st['down_w'], None, 2, 1)
    fea = _conv2d(fea, params['out_proj_w'], None, 1, 0)
    return fea.reshape(fea.shape[0], -1)
